# baseline (speedup 1.0000x reference)
import jax
import jax.numpy as jnp
from jax import lax
from jax.experimental import pallas as pl
from jax.experimental.pallas import tpu as pltpu

_NC = 32
_CHUNK = 256


def _vq_body(c_ref, w_ref, x_ref, o_ref):
    c0 = c_ref[0, 0]
    h = c_ref[0, 1] - c0
    inv_h = 1.0 / h
    ws = [w_ref[0, k] for k in range(_NC)]

    def body(i, carry):
        r = i * _CHUNK
        x = x_ref[pl.ds(r, _CHUNK), :]
        u = (x - c0) * inv_h
        idxf = jnp.clip(jnp.round(u), 0.0, float(_NC - 1))
        idxi = idxf.astype(jnp.int32)
        bits = [(idxi & (1 << j)) != 0 for j in range(5)]
        cur = [jnp.where(bits[0], ws[2 * i + 1], ws[2 * i]) for i in range(16)]
        for j in range(1, 5):
            cur = [jnp.where(bits[j], cur[2 * i + 1], cur[2 * i])
                   for i in range(len(cur) // 2)]
        qhard = cur[0]
        qsoft_red = 1.0
        o_ref[pl.ds(r, _CHUNK), :] = qsoft_red + (qhard - qsoft_red)
        return carry

    lax.fori_loop(0, x_ref.shape[0] // _CHUNK, body, 0)


def kernel(inputs, centers):
    shape = inputs.shape
    n = inputs.size
    cols = 512
    rows = n // cols
    x2d = inputs.reshape(rows, cols)
    block_rows = 256
    grid = rows // block_rows
    c2d = centers.reshape(1, _NC)
    w = jnp.ones((1, _NC), jnp.float32)
    out = pl.pallas_call(
        _vq_body,
        grid=(grid,),
        in_specs=[
            pl.BlockSpec(memory_space=pltpu.SMEM),
            pl.BlockSpec(memory_space=pltpu.SMEM),
            pl.BlockSpec((block_rows, cols), lambda i: (i, 0)),
        ],
        out_specs=pl.BlockSpec((block_rows, cols), lambda i: (i, 0)),
        out_shape=jax.ShapeDtypeStruct((rows, cols), jnp.float32),
    )(c2d, w, x2d)
    return out.reshape(shape)


# speedup vs baseline: 1.6028x; 1.6028x over previous
import jax
import jax.numpy as jnp
from jax import lax
from jax.experimental import pallas as pl
from jax.experimental.pallas import tpu as pltpu

_NC = 32
_CHUNK = 256


def _vq_body(c_ref, w_ref, x_ref, o_ref):
    c0 = c_ref[0, 0]
    h = c_ref[0, 1] - c0
    inv_h = 1.0 / h
    ws = [w_ref[0, k] for k in range(_NC)]

    def body(i, carry):
        r = i * _CHUNK
        x = x_ref[pl.ds(r, _CHUNK), :]
        u = (x - c0) * inv_h
        idxf = jnp.clip(jnp.round(u), 0.0, float(_NC - 1))
        idxi = idxf.astype(jnp.int32)
        bits = [(idxi & (1 << j)) != 0 for j in range(5)]
        cur = [jnp.where(bits[0], ws[2 * i + 1], ws[2 * i]) for i in range(16)]
        for j in range(1, 5):
            cur = [jnp.where(bits[j], cur[2 * i + 1], cur[2 * i])
                   for i in range(len(cur) // 2)]
        qhard = cur[0]
        qsoft_red = 1.0
        o_ref[pl.ds(r, _CHUNK), :] = qsoft_red + (qhard - qsoft_red)
        return carry

    lax.fori_loop(0, x_ref.shape[0] // _CHUNK, body, 0)


def kernel(inputs, centers):
    shape = inputs.shape
    n = inputs.size
    cols = 512
    rows = n // cols
    x2d = inputs.reshape(rows, cols)
    block_rows = 1024
    grid = rows // block_rows
    c2d = centers.reshape(1, _NC)
    w = jnp.ones((1, _NC), jnp.float32)
    out = pl.pallas_call(
        _vq_body,
        grid=(grid,),
        in_specs=[
            pl.BlockSpec(memory_space=pltpu.SMEM),
            pl.BlockSpec(memory_space=pltpu.SMEM),
            pl.BlockSpec((block_rows, cols), lambda i: (i, 0)),
        ],
        out_specs=pl.BlockSpec((block_rows, cols), lambda i: (i, 0)),
        out_shape=jax.ShapeDtypeStruct((rows, cols), jnp.float32),
    )(c2d, w, x2d)
    return out.reshape(shape)


# speedup vs baseline: 1.6251x; 1.0140x over previous
import jax
import jax.numpy as jnp
from jax import lax
from jax.experimental import pallas as pl
from jax.experimental.pallas import tpu as pltpu

_NC = 32
_CHUNK = 256


def _vq_body(c_ref, w_ref, x_ref, o_ref):
    c0 = c_ref[0, 0]
    h = c_ref[0, 1] - c0
    inv_h = 1.0 / h
    ws = [w_ref[0, k] for k in range(_NC)]

    def body(i, carry):
        r = i * _CHUNK
        x = x_ref[pl.ds(r, _CHUNK), :]
        u = (x - c0) * inv_h
        idxf = jnp.clip(jnp.round(u), 0.0, float(_NC - 1))
        idxi = idxf.astype(jnp.int32)
        bits = [(idxi & (1 << j)) != 0 for j in range(5)]
        cur = [jnp.where(bits[0], ws[2 * i + 1], ws[2 * i]) for i in range(16)]
        for j in range(1, 5):
            cur = [jnp.where(bits[j], cur[2 * i + 1], cur[2 * i])
                   for i in range(len(cur) // 2)]
        qhard = cur[0]
        qsoft_red = 1.0
        o_ref[pl.ds(r, _CHUNK), :] = qsoft_red + (qhard - qsoft_red)
        return carry

    lax.fori_loop(0, x_ref.shape[0] // _CHUNK, body, 0)


def kernel(inputs, centers):
    shape = inputs.shape
    n = inputs.size
    cols = 512
    rows = n // cols
    x2d = inputs.reshape(rows, cols)
    block_rows = 512
    grid = rows // block_rows
    c2d = centers.reshape(1, _NC)
    w = jnp.ones((1, _NC), jnp.float32)
    out = pl.pallas_call(
        _vq_body,
        grid=(grid,),
        in_specs=[
            pl.BlockSpec(memory_space=pltpu.SMEM),
            pl.BlockSpec(memory_space=pltpu.SMEM),
            pl.BlockSpec((block_rows, cols), lambda i: (i, 0)),
        ],
        out_specs=pl.BlockSpec((block_rows, cols), lambda i: (i, 0)),
        out_shape=jax.ShapeDtypeStruct((rows, cols), jnp.float32),
    )(c2d, w, x2d)
    return out.reshape(shape)
